# Initial kernel scaffold; baseline (speedup 1.0000x reference)
import functools

import jax
import jax.numpy as jnp
from jax import lax
from jax.experimental import pallas as pl
from jax.experimental.pallas import tpu as pltpu
from jax.experimental.pallas import tpu_sc as plsc

_N_VEC = 1600000
_K_ANG = 3200000
_N_WORKERS = 32
_CHUNK = 2000
_LANES = 16

_HALF_PI = 1.5707963267948966
_PI = 3.141592653589793


def _f32(v):
    return jnp.float32(v)


def _atan01(t):
    z = t * t
    p = _f32(-0.0117212)
    p = p * z + _f32(0.05265332)
    p = p * z + _f32(-0.11643287)
    p = p * z + _f32(0.19354346)
    p = p * z + _f32(-0.33262347)
    p = p * z + _f32(0.99997726)
    return t * p


def _sqrt(y2):
    bi = lax.bitcast_convert_type(y2, jnp.int32)
    bi = jnp.int32(0x5F3759DF) - lax.shift_right_logical(bi, 1)
    g = lax.bitcast_convert_type(bi, jnp.float32)
    half = _f32(0.5) * y2
    g = g * (_f32(1.5) - half * g * g)
    g = g * (_f32(1.5) - half * g * g)
    g = g * (_f32(1.5) - half * g * g)
    return y2 * g


def _angle16(r1_v, r2_v, rows):
    c0 = jnp.zeros((_LANES,), jnp.int32)
    c1 = jnp.full((_LANES,), 1, jnp.int32)
    c2 = jnp.full((_LANES,), 2, jnp.int32)
    a1x = plsc.load_gather(r1_v, [rows, c0])
    a1y = plsc.load_gather(r1_v, [rows, c1])
    a1z = plsc.load_gather(r1_v, [rows, c2])
    a2x = plsc.load_gather(r2_v, [rows, c0])
    a2y = plsc.load_gather(r2_v, [rows, c1])
    a2z = plsc.load_gather(r2_v, [rows, c2])
    x = a1x * a2x + a1y * a2y + a1z * a2z
    cx = a1y * a2z - a1z * a2y
    cy = a1z * a2x - a1x * a2z
    cz = a1x * a2y - a1y * a2x
    y = _sqrt(cx * cx + cy * cy + cz * cz)
    ax = jnp.abs(x) + _f32(1e-30)
    t = y / ax
    inv = t > _f32(1.0)
    tt = jnp.where(inv, _f32(1.0) / t, t)
    a = _atan01(tt)
    a = jnp.where(inv, _f32(_HALF_PI) - a, a)
    return jnp.where(x < _f32(0.0), _f32(_PI) - a, a)


def _make_sc_kernel():
    per_w = _K_ANG // _N_WORKERS
    n_chunks = per_w // _CHUNK
    n_grps = _CHUNK // _LANES
    mesh = plsc.VectorSubcoreMesh(core_axis_name="c", subcore_axis_name="s")

    @functools.partial(
        pl.kernel,
        out_type=jax.ShapeDtypeStruct((_K_ANG,), jnp.float32),
        mesh=mesh,
        scratch_types=[
            pltpu.VMEM((_CHUNK,), jnp.int32),
            pltpu.VMEM((_CHUNK,), jnp.int32),
            pltpu.VMEM((_CHUNK, 3), jnp.float32),
            pltpu.VMEM((_CHUNK, 3), jnp.float32),
            pltpu.VMEM((_CHUNK,), jnp.float32),
            pltpu.SemaphoreType.DMA,
        ],
    )
    def k(vec_hbm, idx0_hbm, idx1_hbm, out_hbm, i0_v, i1_v, r1_v, r2_v, o_v, sem):
        wid = lax.axis_index("s") * 2 + lax.axis_index("c")
        base = wid * per_w
        iota = lax.iota(jnp.int32, _LANES)

        def chunk_body(ci, carry):
            off = base + ci * _CHUNK
            pltpu.sync_copy(idx0_hbm.at[pl.ds(off, _CHUNK)], i0_v)
            pltpu.sync_copy(idx1_hbm.at[pl.ds(off, _CHUNK)], i1_v)
            cp1 = pltpu.async_copy(vec_hbm.at[i0_v], r1_v, sem)
            cp2 = pltpu.async_copy(vec_hbm.at[i1_v], r2_v, sem)
            cp1.wait()
            cp2.wait()

            def grp_body(gi, c2):
                rows = iota + gi * _LANES
                o_v[pl.ds(gi * _LANES, _LANES)] = _angle16(r1_v, r2_v, rows)
                return c2

            lax.fori_loop(0, n_grps, grp_body, 0, unroll=False)
            pltpu.sync_copy(o_v, out_hbm.at[pl.ds(off, _CHUNK)])
            return carry

        lax.fori_loop(0, n_chunks, chunk_body, 0, unroll=False)

    return k


_sc_kernel = _make_sc_kernel()


def kernel(vector, angle_index):
    angles = _sc_kernel(vector, angle_index[0], angle_index[1])
    return angles[:, None]


# speedup vs baseline: 16.7437x; 16.7437x over previous
import functools

import jax
import jax.numpy as jnp
from jax import lax
from jax.experimental import pallas as pl
from jax.experimental.pallas import tpu as pltpu
from jax.experimental.pallas import tpu_sc as plsc

_N_VEC = 1600000
_K_ANG = 3200000
_N_WORKERS = 32
_CHUNK = 2000
_LANES = 16

_HALF_PI = 1.5707963267948966
_PI = 3.141592653589793


def _f32(v):
    return jnp.float32(v)


def _atan01(t):
    z = t * t
    p = _f32(-0.0117212)
    p = p * z + _f32(0.05265332)
    p = p * z + _f32(-0.11643287)
    p = p * z + _f32(0.19354346)
    p = p * z + _f32(-0.33262347)
    p = p * z + _f32(0.99997726)
    return t * p


def _sqrt(y2):
    bi = lax.bitcast_convert_type(y2, jnp.int32)
    bi = jnp.int32(0x5F3759DF) - lax.shift_right_logical(bi, 1)
    g = lax.bitcast_convert_type(bi, jnp.float32)
    half = _f32(0.5) * y2
    g = g * (_f32(1.5) - half * g * g)
    g = g * (_f32(1.5) - half * g * g)
    g = g * (_f32(1.5) - half * g * g)
    return y2 * g


def _angle16(r1_v, r2_v, rows):
    c0 = jnp.zeros((_LANES,), jnp.int32)
    c1 = jnp.full((_LANES,), 1, jnp.int32)
    c2 = jnp.full((_LANES,), 2, jnp.int32)
    a1x = plsc.load_gather(r1_v, [rows, c0])
    a1y = plsc.load_gather(r1_v, [rows, c1])
    a1z = plsc.load_gather(r1_v, [rows, c2])
    a2x = plsc.load_gather(r2_v, [rows, c0])
    a2y = plsc.load_gather(r2_v, [rows, c1])
    a2z = plsc.load_gather(r2_v, [rows, c2])
    x = a1x * a2x + a1y * a2y + a1z * a2z
    cx = a1y * a2z - a1z * a2y
    cy = a1z * a2x - a1x * a2z
    cz = a1x * a2y - a1y * a2x
    y = _sqrt(cx * cx + cy * cy + cz * cz)
    ax = jnp.abs(x) + _f32(1e-30)
    t = y / ax
    inv = t > _f32(1.0)
    tt = jnp.where(inv, _f32(1.0) / t, t)
    a = _atan01(tt)
    a = jnp.where(inv, _f32(_HALF_PI) - a, a)
    return jnp.where(x < _f32(0.0), _f32(_PI) - a, a)


def _make_sc_kernel():
    per_w = _K_ANG // _N_WORKERS
    n_chunks = per_w // _CHUNK
    n_grps = _CHUNK // _LANES
    mesh = plsc.VectorSubcoreMesh(core_axis_name="c", subcore_axis_name="s")

    @functools.partial(
        pl.kernel,
        out_type=jax.ShapeDtypeStruct((_K_ANG,), jnp.float32),
        mesh=mesh,
        compiler_params=pltpu.CompilerParams(
            needs_layout_passes=False, use_tc_tiling_on_sc=False
        ),
        scratch_types=[
            pltpu.VMEM((_CHUNK,), jnp.int32),
            pltpu.VMEM((_CHUNK,), jnp.int32),
            pltpu.VMEM((_CHUNK, 3), jnp.float32),
            pltpu.VMEM((_CHUNK, 3), jnp.float32),
            pltpu.VMEM((_CHUNK,), jnp.float32),
            pltpu.SemaphoreType.DMA,
        ],
    )
    def k(vec_hbm, idx0_hbm, idx1_hbm, out_hbm, i0_v, i1_v, r1_v, r2_v, o_v, sem):
        wid = lax.axis_index("s") * 2 + lax.axis_index("c")
        base = wid * per_w
        iota = lax.iota(jnp.int32, _LANES)

        def chunk_body(ci, carry):
            off = base + ci * _CHUNK
            pltpu.sync_copy(idx0_hbm.at[pl.ds(off, _CHUNK)], i0_v)
            pltpu.sync_copy(idx1_hbm.at[pl.ds(off, _CHUNK)], i1_v)
            cp1 = pltpu.async_copy(vec_hbm.at[i0_v], r1_v, sem)
            cp2 = pltpu.async_copy(vec_hbm.at[i1_v], r2_v, sem)
            cp1.wait()
            cp2.wait()

            def grp_body(gi, c2):
                rows = iota + gi * _LANES
                o_v[pl.ds(gi * _LANES, _LANES)] = _angle16(r1_v, r2_v, rows)
                return c2

            lax.fori_loop(0, n_grps, grp_body, 0, unroll=False)
            pltpu.sync_copy(o_v, out_hbm.at[pl.ds(off, _CHUNK)])
            return carry

        lax.fori_loop(0, n_chunks, chunk_body, 0, unroll=False)

    return k


_sc_kernel = _make_sc_kernel()


def kernel(vector, angle_index):
    angles = _sc_kernel(vector, angle_index[0], angle_index[1])
    return angles[:, None]


# speedup vs baseline: 107.4597x; 6.4179x over previous
import functools

import jax
import jax.numpy as jnp
from jax import lax
from jax.experimental import pallas as pl
from jax.experimental.pallas import tpu as pltpu
from jax.experimental.pallas import tpu_sc as plsc

_K_ANG = 3200000
_N_WORKERS = 32
_CHUNK = 2000
_LANES = 16

_HALF_PI = 1.5707963267948966
_PI = 3.141592653589793


def _f32(v):
    return jnp.float32(v)


def _atan01(t):
    z = t * t
    p = _f32(-0.0117212)
    p = p * z + _f32(0.05265332)
    p = p * z + _f32(-0.11643287)
    p = p * z + _f32(0.19354346)
    p = p * z + _f32(-0.33262347)
    p = p * z + _f32(0.99997726)
    return t * p


def _sqrt(y2):
    bi = lax.bitcast_convert_type(y2, jnp.int32)
    bi = jnp.int32(0x5F3759DF) - lax.shift_right_logical(bi, 1)
    g = lax.bitcast_convert_type(bi, jnp.float32)
    half = _f32(0.5) * y2
    g = g * (_f32(1.5) - half * g * g)
    g = g * (_f32(1.5) - half * g * g)
    g = g * (_f32(1.5) - half * g * g)
    return y2 * g


def _angle16(a1x, a1y, a1z, a2x, a2y, a2z):
    x = a1x * a2x + a1y * a2y + a1z * a2z
    cx = a1y * a2z - a1z * a2y
    cy = a1z * a2x - a1x * a2z
    cz = a1x * a2y - a1y * a2x
    y = _sqrt(cx * cx + cy * cy + cz * cz)
    ax = jnp.abs(x) + _f32(1e-30)
    t = y / ax
    inv = t > _f32(1.0)
    tt = jnp.where(inv, _f32(1.0) / t, t)
    a = _atan01(tt)
    a = jnp.where(inv, _f32(_HALF_PI) - a, a)
    return jnp.where(x < _f32(0.0), _f32(_PI) - a, a)


def _make_sc_kernel():
    per_w = _K_ANG // _N_WORKERS
    n_chunks = per_w // _CHUNK
    n_grps = _CHUNK // _LANES
    mesh = plsc.VectorSubcoreMesh(core_axis_name="c", subcore_axis_name="s")

    @functools.partial(
        pl.kernel,
        out_type=jax.ShapeDtypeStruct((_K_ANG,), jnp.float32),
        mesh=mesh,
        compiler_params=pltpu.CompilerParams(needs_layout_passes=False),
        scratch_types=[
            pltpu.VMEM((_CHUNK,), jnp.int32),
            pltpu.VMEM((_CHUNK,), jnp.int32),
            pltpu.VMEM((_CHUNK,), jnp.float32),
            pltpu.VMEM((_CHUNK,), jnp.float32),
            pltpu.VMEM((_CHUNK,), jnp.float32),
            pltpu.VMEM((_CHUNK,), jnp.float32),
            pltpu.VMEM((_CHUNK,), jnp.float32),
            pltpu.VMEM((_CHUNK,), jnp.float32),
            pltpu.VMEM((_CHUNK,), jnp.float32),
            pltpu.SemaphoreType.DMA,
        ],
    )
    def k(vx_hbm, vy_hbm, vz_hbm, idx0_hbm, idx1_hbm, out_hbm,
          i0_v, i1_v, g1x, g1y, g1z, g2x, g2y, g2z, o_v, sem):
        wid = lax.axis_index("s") * 2 + lax.axis_index("c")
        base = wid * per_w

        def chunk_body(ci, carry):
            off = base + ci * _CHUNK
            pltpu.sync_copy(idx0_hbm.at[pl.ds(off, _CHUNK)], i0_v)
            pltpu.sync_copy(idx1_hbm.at[pl.ds(off, _CHUNK)], i1_v)
            cps = [
                pltpu.async_copy(vx_hbm.at[i0_v], g1x, sem),
                pltpu.async_copy(vy_hbm.at[i0_v], g1y, sem),
                pltpu.async_copy(vz_hbm.at[i0_v], g1z, sem),
                pltpu.async_copy(vx_hbm.at[i1_v], g2x, sem),
                pltpu.async_copy(vy_hbm.at[i1_v], g2y, sem),
                pltpu.async_copy(vz_hbm.at[i1_v], g2z, sem),
            ]
            for cp in cps:
                cp.wait()

            def grp_body(gi, c2):
                s = pl.ds(gi * _LANES, _LANES)
                o_v[s] = _angle16(g1x[s], g1y[s], g1z[s],
                                  g2x[s], g2y[s], g2z[s])
                return c2

            lax.fori_loop(0, n_grps, grp_body, 0, unroll=False)
            pltpu.sync_copy(o_v, out_hbm.at[pl.ds(off, _CHUNK)])
            return carry

        lax.fori_loop(0, n_chunks, chunk_body, 0, unroll=False)

    return k


_sc_kernel = _make_sc_kernel()


def kernel(vector, angle_index):
    vx = vector[:, 0]
    vy = vector[:, 1]
    vz = vector[:, 2]
    angles = _sc_kernel(vx, vy, vz, angle_index[0], angle_index[1])
    return angles[:, None]


# speedup vs baseline: 128.6450x; 1.1971x over previous
import functools

import jax
import jax.numpy as jnp
from jax import lax
from jax.experimental import pallas as pl
from jax.experimental.pallas import tpu as pltpu
from jax.experimental.pallas import tpu_sc as plsc

_K_ANG = 3200000
_N_WORKERS = 32
_CHUNK = 2000
_LANES = 16

_HALF_PI = 1.5707963267948966
_PI = 3.141592653589793


def _f32(v):
    return jnp.float32(v)


def _atan01(t):
    z = t * t
    p = _f32(-0.0117212)
    p = p * z + _f32(0.05265332)
    p = p * z + _f32(-0.11643287)
    p = p * z + _f32(0.19354346)
    p = p * z + _f32(-0.33262347)
    p = p * z + _f32(0.99997726)
    return t * p


def _sqrt(y2):
    bi = lax.bitcast_convert_type(y2, jnp.int32)
    bi = jnp.int32(0x5F3759DF) - lax.shift_right_logical(bi, 1)
    g = lax.bitcast_convert_type(bi, jnp.float32)
    half = _f32(0.5) * y2
    g = g * (_f32(1.5) - half * g * g)
    g = g * (_f32(1.5) - half * g * g)
    g = g * (_f32(1.5) - half * g * g)
    return y2 * g


def _angle16(a1x, a1y, a1z, a2x, a2y, a2z):
    x = a1x * a2x + a1y * a2y + a1z * a2z
    cx = a1y * a2z - a1z * a2y
    cy = a1z * a2x - a1x * a2z
    cz = a1x * a2y - a1y * a2x
    y = _sqrt(cx * cx + cy * cy + cz * cz)
    ax = jnp.abs(x) + _f32(1e-30)
    t = y / ax
    inv = t > _f32(1.0)
    tt = jnp.where(inv, _f32(1.0) / t, t)
    a = _atan01(tt)
    a = jnp.where(inv, _f32(_HALF_PI) - a, a)
    return jnp.where(x < _f32(0.0), _f32(_PI) - a, a)


def _make_sc_kernel():
    per_w = _K_ANG // _N_WORKERS
    n_chunks = per_w // _CHUNK
    n_grps = _CHUNK // _LANES
    assert n_chunks % 2 == 0
    mesh = plsc.VectorSubcoreMesh(core_axis_name="c", subcore_axis_name="s")

    buf = lambda dt: pltpu.VMEM((_CHUNK,), dt)
    scratch = []
    for _b in range(2):
        scratch += [buf(jnp.int32), buf(jnp.int32)]
        scratch += [buf(jnp.float32) for _ in range(6)]
        scratch += [buf(jnp.float32)]
        scratch += [pltpu.SemaphoreType.DMA,
                    pltpu.SemaphoreType.DMA]

    @functools.partial(
        pl.kernel,
        out_type=jax.ShapeDtypeStruct((_K_ANG,), jnp.float32),
        mesh=mesh,
        compiler_params=pltpu.CompilerParams(needs_layout_passes=False),
        scratch_types=scratch,
    )
    def k(vx_hbm, vy_hbm, vz_hbm, idx0_hbm, idx1_hbm, out_hbm, *bufs):
        slots = [bufs[i * 11:(i + 1) * 11] for i in range(2)]
        wid = lax.axis_index("s") * 2 + lax.axis_index("c")
        base = wid * per_w

        def issue(b, off):
            i0_v, i1_v, g1x, g1y, g1z, g2x, g2y, g2z, _o, sem, _so = slots[b]
            pltpu.sync_copy(idx0_hbm.at[pl.ds(off, _CHUNK)], i0_v)
            pltpu.sync_copy(idx1_hbm.at[pl.ds(off, _CHUNK)], i1_v)
            pltpu.async_copy(vx_hbm.at[i0_v], g1x, sem)
            pltpu.async_copy(vy_hbm.at[i0_v], g1y, sem)
            pltpu.async_copy(vz_hbm.at[i0_v], g1z, sem)
            pltpu.async_copy(vx_hbm.at[i1_v], g2x, sem)
            pltpu.async_copy(vy_hbm.at[i1_v], g2y, sem)
            pltpu.async_copy(vz_hbm.at[i1_v], g2z, sem)

        def drain_gathers(b):
            i0_v, _i1, g1x, g1y, g1z, g2x, g2y, g2z, _o, sem, _so = slots[b]
            for g in (g1x, g1y, g1z, g2x, g2y, g2z):
                pltpu.make_async_copy(vx_hbm.at[i0_v], g, sem).wait()

        def compute_and_store(b, ci):
            (_i0, _i1, g1x, g1y, g1z, g2x, g2y, g2z, o_v, _sem,
             semo) = slots[b]
            off = base + ci * _CHUNK

            def grp_body(gi, c2):
                s = pl.ds(gi * _LANES, _LANES)
                o_v[s] = _angle16(g1x[s], g1y[s], g1z[s],
                                  g2x[s], g2y[s], g2z[s])
                return c2

            @pl.when(ci >= 2)
            def _():
                pltpu.make_async_copy(
                    o_v, out_hbm.at[pl.ds(off, _CHUNK)], semo).wait()

            lax.fori_loop(0, n_grps, grp_body, 0, unroll=False)
            pltpu.async_copy(o_v, out_hbm.at[pl.ds(off, _CHUNK)], semo)

        issue(0, base)

        def pair_body(pi, carry):
            for b in range(2):
                ci = pi * 2 + b

                @pl.when(ci + 1 < n_chunks)
                def _():
                    issue(1 - b, base + (ci + 1) * _CHUNK)

                drain_gathers(b)
                compute_and_store(b, ci)
            return carry

        lax.fori_loop(0, n_chunks // 2, pair_body, 0, unroll=False)
        for b in range(2):
            o_v, semo = slots[b][8], slots[b][10]
            pltpu.make_async_copy(
                o_v, out_hbm.at[pl.ds(base, _CHUNK)], semo).wait()

    return k


_sc_kernel = _make_sc_kernel()


def kernel(vector, angle_index):
    vx = vector[:, 0]
    vy = vector[:, 1]
    vz = vector[:, 2]
    angles = _sc_kernel(vx, vy, vz, angle_index[0], angle_index[1])
    return angles[:, None]


# speedup vs baseline: 128.7667x; 1.0009x over previous
import functools

import jax
import jax.numpy as jnp
from jax import lax
from jax.experimental import pallas as pl
from jax.experimental.pallas import tpu as pltpu
from jax.experimental.pallas import tpu_sc as plsc

_K_ANG = 3200000
_N_WORKERS = 32
_CHUNK = 5000
_LANES = 16

_HALF_PI = 1.5707963267948966
_PI = 3.141592653589793


def _f32(v):
    return jnp.float32(v)


def _atan01(t):
    z = t * t
    p = _f32(-0.0117212)
    p = p * z + _f32(0.05265332)
    p = p * z + _f32(-0.11643287)
    p = p * z + _f32(0.19354346)
    p = p * z + _f32(-0.33262347)
    p = p * z + _f32(0.99997726)
    return t * p


def _sqrt(y2):
    bi = lax.bitcast_convert_type(y2, jnp.int32)
    bi = jnp.int32(0x5F3759DF) - lax.shift_right_logical(bi, 1)
    g = lax.bitcast_convert_type(bi, jnp.float32)
    half = _f32(0.5) * y2
    g = g * (_f32(1.5) - half * g * g)
    g = g * (_f32(1.5) - half * g * g)
    g = g * (_f32(1.5) - half * g * g)
    return y2 * g


def _angle16(a1x, a1y, a1z, a2x, a2y, a2z):
    x = a1x * a2x + a1y * a2y + a1z * a2z
    cx = a1y * a2z - a1z * a2y
    cy = a1z * a2x - a1x * a2z
    cz = a1x * a2y - a1y * a2x
    y = _sqrt(cx * cx + cy * cy + cz * cz)
    ax = jnp.abs(x) + _f32(1e-30)
    t = y / ax
    inv = t > _f32(1.0)
    tt = jnp.where(inv, _f32(1.0) / t, t)
    a = _atan01(tt)
    a = jnp.where(inv, _f32(_HALF_PI) - a, a)
    return jnp.where(x < _f32(0.0), _f32(_PI) - a, a)


def _make_sc_kernel():
    per_w = _K_ANG // _N_WORKERS
    n_chunks = per_w // _CHUNK
    n_grps = _CHUNK // _LANES
    assert n_chunks % 2 == 0
    mesh = plsc.VectorSubcoreMesh(core_axis_name="c", subcore_axis_name="s")

    buf = lambda dt: pltpu.VMEM((_CHUNK,), dt)
    scratch = []
    for _b in range(2):
        scratch += [buf(jnp.int32), buf(jnp.int32)]
        scratch += [buf(jnp.float32) for _ in range(6)]
        scratch += [buf(jnp.float32)]
        scratch += [pltpu.SemaphoreType.DMA,
                    pltpu.SemaphoreType.DMA]

    @functools.partial(
        pl.kernel,
        out_type=jax.ShapeDtypeStruct((_K_ANG,), jnp.float32),
        mesh=mesh,
        compiler_params=pltpu.CompilerParams(needs_layout_passes=False),
        scratch_types=scratch,
    )
    def k(vx_hbm, vy_hbm, vz_hbm, idx0_hbm, idx1_hbm, out_hbm, *bufs):
        slots = [bufs[i * 11:(i + 1) * 11] for i in range(2)]
        wid = lax.axis_index("s") * 2 + lax.axis_index("c")
        base = wid * per_w

        def issue(b, off):
            i0_v, i1_v, g1x, g1y, g1z, g2x, g2y, g2z, _o, sem, _so = slots[b]
            pltpu.sync_copy(idx0_hbm.at[pl.ds(off, _CHUNK)], i0_v)
            pltpu.sync_copy(idx1_hbm.at[pl.ds(off, _CHUNK)], i1_v)
            pltpu.async_copy(vx_hbm.at[i0_v], g1x, sem)
            pltpu.async_copy(vy_hbm.at[i0_v], g1y, sem)
            pltpu.async_copy(vz_hbm.at[i0_v], g1z, sem)
            pltpu.async_copy(vx_hbm.at[i1_v], g2x, sem)
            pltpu.async_copy(vy_hbm.at[i1_v], g2y, sem)
            pltpu.async_copy(vz_hbm.at[i1_v], g2z, sem)

        def drain_gathers(b):
            i0_v, _i1, g1x, g1y, g1z, g2x, g2y, g2z, _o, sem, _so = slots[b]
            for g in (g1x, g1y, g1z, g2x, g2y, g2z):
                pltpu.make_async_copy(vx_hbm.at[i0_v], g, sem).wait()

        def compute_and_store(b, ci):
            (_i0, _i1, g1x, g1y, g1z, g2x, g2y, g2z, o_v, _sem,
             semo) = slots[b]
            off = base + ci * _CHUNK

            def grp_body(gi, c2):
                s = pl.ds(gi * _LANES, _LANES)
                o_v[s] = _angle16(g1x[s], g1y[s], g1z[s],
                                  g2x[s], g2y[s], g2z[s])
                return c2

            @pl.when(ci >= 2)
            def _():
                pltpu.make_async_copy(
                    o_v, out_hbm.at[pl.ds(off, _CHUNK)], semo).wait()

            lax.fori_loop(0, n_grps, grp_body, 0, unroll=False)
            pltpu.async_copy(o_v, out_hbm.at[pl.ds(off, _CHUNK)], semo)

        issue(0, base)

        def pair_body(pi, carry):
            for b in range(2):
                ci = pi * 2 + b

                @pl.when(ci + 1 < n_chunks)
                def _():
                    issue(1 - b, base + (ci + 1) * _CHUNK)

                drain_gathers(b)
                compute_and_store(b, ci)
            return carry

        lax.fori_loop(0, n_chunks // 2, pair_body, 0, unroll=False)
        for b in range(2):
            o_v, semo = slots[b][8], slots[b][10]
            pltpu.make_async_copy(
                o_v, out_hbm.at[pl.ds(base, _CHUNK)], semo).wait()

    return k


_sc_kernel = _make_sc_kernel()


def kernel(vector, angle_index):
    vx = vector[:, 0]
    vy = vector[:, 1]
    vz = vector[:, 2]
    angles = _sc_kernel(vx, vy, vz, angle_index[0], angle_index[1])
    return angles[:, None]
